# baseline (speedup 1.0000x reference)
import jax
import jax.numpy as jnp
from jax.experimental import pallas as pl
from jax.experimental.pallas import tpu as pltpu

ALPHA = 1.702
LIMIT = 7.0

CI = 512


def _ffn_kernel(idx_ref, t_ref, w_ref, bg_ref, bl_ref, wd_ref,
                bd_ref, out_ref, y_acc):
    e = pl.program_id(0)
    c = pl.program_id(1)
    n_c = pl.num_programs(1)

    x = t_ref[...].astype(jnp.bfloat16)
    w = w_ref[0].astype(jnp.bfloat16)

    dims = (((1,), (1,)), ((), ()))
    h = jax.lax.dot_general(x, w, dims,
                            preferred_element_type=jnp.float32)
    B, CI2 = h.shape
    even = jax.lax.broadcasted_iota(jnp.int32, (B, 64), 1) * 2
    parts_g, parts_l = [], []
    for m in range(CI2 // 128):
        blk = h[:, 128 * m:128 * (m + 1)]
        parts_g.append(jnp.take_along_axis(blk, even, axis=1))
        parts_l.append(jnp.take_along_axis(blk, even + 1, axis=1))
    h_glu = jnp.concatenate(parts_g, axis=1) + bg_ref[0]
    h_lin = jnp.concatenate(parts_l, axis=1) + bl_ref[0]

    h_glu = jnp.minimum(h_glu, LIMIT)
    h_lin = jnp.clip(h_lin, -LIMIT, LIMIT)
    act = h_glu * jax.nn.sigmoid(ALPHA * h_glu) * (h_lin + 1.0)

    wd = wd_ref[0].astype(jnp.bfloat16)
    y = jax.lax.dot_general(act.astype(jnp.bfloat16), wd, dims,
                            preferred_element_type=jnp.float32)

    @pl.when(c == 0)
    def _():
        y_acc[...] = y

    @pl.when(c != 0)
    def _():
        y_acc[...] = y_acc[...] + y

    @pl.when(c == n_c - 1)
    def _():
        yv = y_acc[...] + bd_ref[0]
        for k in range(idx_ref.shape[1]):
            sel = (idx_ref[:, k:k + 1] == e)
            contrib = jnp.where(sel, yv, 0.0)

            @pl.when(e == 0)
            def _():
                out_ref[:, k, :] = contrib

            @pl.when(e != 0)
            def _():
                out_ref[:, k, :] = out_ref[:, k, :] + contrib


def kernel(t, expert_indices, gate_up_proj, gate_up_proj_bias, down_proj,
           down_proj_bias):
    B, H = t.shape
    TOPK = expert_indices.shape[1]
    E, I2, _ = gate_up_proj.shape
    I = I2 // 2
    n_chunks = I // CI

    bias_glu = gate_up_proj_bias[:, 0::2].reshape(E, 1, I)
    bias_lin = gate_up_proj_bias[:, 1::2].reshape(E, 1, I)
    dbias = down_proj_bias.reshape(E, 1, H)
    idx = expert_indices.astype(jnp.int32)

    grid = (E, n_chunks)
    out = pl.pallas_call(
        _ffn_kernel,
        grid=grid,
        in_specs=[
            pl.BlockSpec((B, TOPK), lambda e, c: (0, 0)),
            pl.BlockSpec((B, H), lambda e, c: (0, 0)),
            pl.BlockSpec((1, 2 * CI, H), lambda e, c: (e, c, 0)),
            pl.BlockSpec((1, 1, CI), lambda e, c: (e, 0, c)),
            pl.BlockSpec((1, 1, CI), lambda e, c: (e, 0, c)),
            pl.BlockSpec((1, H, CI), lambda e, c: (e, 0, c)),
            pl.BlockSpec((1, 1, H), lambda e, c: (e, 0, 0)),
        ],
        out_specs=pl.BlockSpec((B, TOPK, H), lambda e, c: (0, 0, 0)),
        out_shape=jax.ShapeDtypeStruct((B, TOPK, H), jnp.float32),
        scratch_shapes=[pltpu.VMEM((B, H), jnp.float32)],
        compiler_params=pltpu.CompilerParams(
            dimension_semantics=("arbitrary", "arbitrary")),
    )(idx, t, gate_up_proj, bias_glu, bias_lin, down_proj, dbias)
    return out


# speedup vs baseline: 1.1188x; 1.1188x over previous
import jax
import jax.numpy as jnp
from jax.experimental import pallas as pl
from jax.experimental.pallas import tpu as pltpu

ALPHA = 1.702
LIMIT = 7.0

CI = 1024


def _deinterleave(h):
    B, n = h.shape
    even = jax.lax.broadcasted_iota(jnp.int32, (B, 64), 1) * 2
    parts_g, parts_l = [], []
    for m in range(n // 128):
        blk = h[:, 128 * m:128 * (m + 1)]
        parts_g.append(jnp.take_along_axis(blk, even, axis=1))
        parts_l.append(jnp.take_along_axis(blk, even + 1, axis=1))
    return jnp.concatenate(parts_g, axis=1), jnp.concatenate(parts_l, axis=1)


def _ffn_kernel(idx_ref, t_ref, wa_ref, wb_ref, bg_ref, bl_ref, wd_ref,
                bd_ref, out_ref, y_acc):
    e = pl.program_id(0)
    c = pl.program_id(1)
    n_c = pl.num_programs(1)

    x = t_ref[...].astype(jnp.bfloat16)
    dims = (((1,), (1,)), ((), ()))
    ha = jax.lax.dot_general(x, wa_ref[0].astype(jnp.bfloat16), dims,
                             preferred_element_type=jnp.float32)
    hb = jax.lax.dot_general(x, wb_ref[0].astype(jnp.bfloat16), dims,
                             preferred_element_type=jnp.float32)
    ga, la = _deinterleave(ha)
    gb, lb = _deinterleave(hb)
    h_glu = jnp.concatenate([ga, gb], axis=1) + bg_ref[0]
    h_lin = jnp.concatenate([la, lb], axis=1) + bl_ref[0]

    h_glu = jnp.minimum(h_glu, LIMIT)
    h_lin = jnp.clip(h_lin, -LIMIT, LIMIT)
    act = h_glu * jax.nn.sigmoid(ALPHA * h_glu) * (h_lin + 1.0)

    wd = wd_ref[0].astype(jnp.bfloat16)
    y = jax.lax.dot_general(act.astype(jnp.bfloat16), wd, dims,
                            preferred_element_type=jnp.float32)

    @pl.when(c == 0)
    def _():
        y_acc[...] = y

    @pl.when(c != 0)
    def _():
        y_acc[...] = y_acc[...] + y

    @pl.when(c == n_c - 1)
    def _():
        yv = y_acc[...] + bd_ref[0]
        for k in range(idx_ref.shape[1]):
            sel = (idx_ref[:, k:k + 1] == e)
            contrib = jnp.where(sel, yv, 0.0)

            @pl.when(e == 0)
            def _():
                out_ref[:, k, :] = contrib

            @pl.when(e != 0)
            def _():
                out_ref[:, k, :] = out_ref[:, k, :] + contrib


def kernel(t, expert_indices, gate_up_proj, gate_up_proj_bias, down_proj,
           down_proj_bias):
    B, H = t.shape
    TOPK = expert_indices.shape[1]
    E, I2, _ = gate_up_proj.shape
    I = I2 // 2
    n_chunks = I // CI

    bias_glu = gate_up_proj_bias[:, 0::2].reshape(E, 1, I)
    bias_lin = gate_up_proj_bias[:, 1::2].reshape(E, 1, I)
    dbias = down_proj_bias.reshape(E, 1, H)
    idx = expert_indices.astype(jnp.int32)

    grid = (E, n_chunks)
    out = pl.pallas_call(
        _ffn_kernel,
        grid=grid,
        in_specs=[
            pl.BlockSpec((B, TOPK), lambda e, c: (0, 0)),
            pl.BlockSpec((B, H), lambda e, c: (0, 0)),
            pl.BlockSpec((1, CI, H), lambda e, c: (e, 2 * c, 0)),
            pl.BlockSpec((1, CI, H), lambda e, c: (e, 2 * c + 1, 0)),
            pl.BlockSpec((1, 1, CI), lambda e, c: (e, 0, c)),
            pl.BlockSpec((1, 1, CI), lambda e, c: (e, 0, c)),
            pl.BlockSpec((1, H, CI), lambda e, c: (e, 0, c)),
            pl.BlockSpec((1, 1, H), lambda e, c: (e, 0, 0)),
        ],
        out_specs=pl.BlockSpec((B, TOPK, H), lambda e, c: (0, 0, 0)),
        out_shape=jax.ShapeDtypeStruct((B, TOPK, H), jnp.float32),
        scratch_shapes=[pltpu.VMEM((B, H), jnp.float32)],
        compiler_params=pltpu.CompilerParams(
            dimension_semantics=("arbitrary", "arbitrary")),
    )(idx, t, gate_up_proj, gate_up_proj, bias_glu, bias_lin, down_proj,
      dbias)
    return out


# speedup vs baseline: 1.2152x; 1.0861x over previous
import jax
import jax.numpy as jnp
from jax.experimental import pallas as pl
from jax.experimental.pallas import tpu as pltpu

ALPHA = 1.702
LIMIT = 7.0

CI = 1024


def _deinterleave(h):
    B, n = h.shape
    even = jax.lax.broadcasted_iota(jnp.int32, (B, 64), 1) * 2
    parts_g, parts_l = [], []
    for m in range(n // 128):
        blk = h[:, 128 * m:128 * (m + 1)]
        parts_g.append(jnp.take_along_axis(blk, even, axis=1))
        parts_l.append(jnp.take_along_axis(blk, even + 1, axis=1))
    return jnp.concatenate(parts_g, axis=1), jnp.concatenate(parts_l, axis=1)


def _ffn_kernel(idx_ref, t_ref, wa_ref, wb_ref, ba_ref, bb_ref, wd_ref,
                bd_ref, out_ref, y_acc):
    e = pl.program_id(0)
    c = pl.program_id(1)
    n_c = pl.num_programs(1)

    x = t_ref[...].astype(jnp.bfloat16)
    dims = (((1,), (1,)), ((), ()))
    ha = jax.lax.dot_general(x, wa_ref[0].astype(jnp.bfloat16), dims,
                             preferred_element_type=jnp.float32)
    hb = jax.lax.dot_general(x, wb_ref[0].astype(jnp.bfloat16), dims,
                             preferred_element_type=jnp.float32)
    ga, la = _deinterleave(ha + ba_ref[0, 0])
    gb, lb = _deinterleave(hb + bb_ref[0, 0])
    h_glu = jnp.concatenate([ga, gb], axis=1)
    h_lin = jnp.concatenate([la, lb], axis=1)

    h_glu = jnp.minimum(h_glu, LIMIT)
    h_lin = jnp.clip(h_lin, -LIMIT, LIMIT)
    act = h_glu * jax.nn.sigmoid(ALPHA * h_glu) * (h_lin + 1.0)

    wd = wd_ref[0].astype(jnp.bfloat16)
    y = jax.lax.dot_general(act.astype(jnp.bfloat16), wd, dims,
                            preferred_element_type=jnp.float32)

    @pl.when(c == 0)
    def _():
        y_acc[...] = y

    @pl.when(c != 0)
    def _():
        y_acc[...] = y_acc[...] + y

    @pl.when(c == n_c - 1)
    def _():
        yv = y_acc[...] + bd_ref[0]
        for k in range(idx_ref.shape[1]):
            sel = (idx_ref[:, k:k + 1] == e)
            contrib = jnp.where(sel, yv, 0.0)

            @pl.when(e == 0)
            def _():
                out_ref[:, k, :] = contrib

            @pl.when(e != 0)
            def _():
                out_ref[:, k, :] = out_ref[:, k, :] + contrib


def kernel(t, expert_indices, gate_up_proj, gate_up_proj_bias, down_proj,
           down_proj_bias):
    B, H = t.shape
    TOPK = expert_indices.shape[1]
    E, I2, _ = gate_up_proj.shape
    I = I2 // 2
    n_chunks = I // CI

    gbias = gate_up_proj_bias.reshape(E, I2 // CI, 1, CI)
    dbias = down_proj_bias.reshape(E, 1, H)
    idx = (expert_indices if expert_indices.dtype == jnp.int32
           else expert_indices.astype(jnp.int32))

    grid = (E, n_chunks)
    out = pl.pallas_call(
        _ffn_kernel,
        grid=grid,
        in_specs=[
            pl.BlockSpec((B, TOPK), lambda e, c: (0, 0)),
            pl.BlockSpec((B, H), lambda e, c: (0, 0)),
            pl.BlockSpec((1, CI, H), lambda e, c: (e, 2 * c, 0)),
            pl.BlockSpec((1, CI, H), lambda e, c: (e, 2 * c + 1, 0)),
            pl.BlockSpec((1, 1, 1, CI), lambda e, c: (e, 2 * c, 0, 0)),
            pl.BlockSpec((1, 1, 1, CI), lambda e, c: (e, 2 * c + 1, 0, 0)),
            pl.BlockSpec((1, H, CI), lambda e, c: (e, 0, c)),
            pl.BlockSpec((1, 1, H), lambda e, c: (e, 0, 0)),
        ],
        out_specs=pl.BlockSpec((B, TOPK, H), lambda e, c: (0, 0, 0)),
        out_shape=jax.ShapeDtypeStruct((B, TOPK, H), jnp.float32),
        scratch_shapes=[pltpu.VMEM((B, H), jnp.float32)],
        compiler_params=pltpu.CompilerParams(
            dimension_semantics=("arbitrary", "arbitrary")),
    )(idx, t, gate_up_proj, gate_up_proj, gbias, gbias, down_proj, dbias)
    return out


# speedup vs baseline: 1.3176x; 1.0843x over previous
import jax
import jax.numpy as jnp
from jax.experimental import pallas as pl
from jax.experimental.pallas import tpu as pltpu

ALPHA = 1.702
LIMIT = 7.0

CI = 1024


def _deinterleave(h):
    B, n = h.shape
    even = jax.lax.broadcasted_iota(jnp.int32, (B, 64), 1) * 2
    parts_g, parts_l = [], []
    for m in range(n // 128):
        blk = h[:, 128 * m:128 * (m + 1)]
        parts_g.append(jnp.take_along_axis(blk, even, axis=1))
        parts_l.append(jnp.take_along_axis(blk, even + 1, axis=1))
    return jnp.concatenate(parts_g, axis=1), jnp.concatenate(parts_l, axis=1)


def _ffn_kernel(idx_ref, t_ref, wa_ref, wb_ref, bg_ref, wd_ref,
                bd_ref, out_ref, y_acc):
    e = pl.program_id(0)
    c = pl.program_id(1)
    n_c = pl.num_programs(1)

    x = t_ref[...].astype(jnp.bfloat16)
    dims = (((1,), (1,)), ((), ()))
    ha = jax.lax.dot_general(x, wa_ref[0].astype(jnp.bfloat16), dims,
                             preferred_element_type=jnp.float32)
    hb = jax.lax.dot_general(x, wb_ref[0].astype(jnp.bfloat16), dims,
                             preferred_element_type=jnp.float32)
    ba = bg_ref[pl.ds(e, 1), pl.ds(2 * c * CI, CI)]
    bb = bg_ref[pl.ds(e, 1), pl.ds((2 * c + 1) * CI, CI)]
    ga, la = _deinterleave(ha + ba)
    gb, lb = _deinterleave(hb + bb)
    h_glu = jnp.concatenate([ga, gb], axis=1)
    h_lin = jnp.concatenate([la, lb], axis=1)

    h_glu = jnp.minimum(h_glu, LIMIT)
    h_lin = jnp.clip(h_lin, -LIMIT, LIMIT)
    act = h_glu * jax.nn.sigmoid(ALPHA * h_glu) * (h_lin + 1.0)

    wd = wd_ref[0].astype(jnp.bfloat16)
    y = jax.lax.dot_general(act.astype(jnp.bfloat16), wd, dims,
                            preferred_element_type=jnp.float32)

    @pl.when(c == 0)
    def _():
        y_acc[...] = y

    @pl.when(c != 0)
    def _():
        y_acc[...] = y_acc[...] + y

    @pl.when(c == n_c - 1)
    def _():
        yv = y_acc[...] + bd_ref[pl.ds(e, 1), :]
        for k in range(idx_ref.shape[1]):
            sel = (idx_ref[:, k:k + 1] == e)
            contrib = jnp.where(sel, yv, 0.0)

            @pl.when(e == 0)
            def _():
                out_ref[:, k, :] = contrib

            @pl.when(e != 0)
            def _():
                out_ref[:, k, :] = out_ref[:, k, :] + contrib


def kernel(t, expert_indices, gate_up_proj, gate_up_proj_bias, down_proj,
           down_proj_bias):
    B, H = t.shape
    TOPK = expert_indices.shape[1]
    E, I2, _ = gate_up_proj.shape
    I = I2 // 2
    n_chunks = I // CI

    idx = (expert_indices if expert_indices.dtype == jnp.int32
           else expert_indices.astype(jnp.int32))

    grid = (E, n_chunks)
    out = pl.pallas_call(
        _ffn_kernel,
        grid=grid,
        in_specs=[
            pl.BlockSpec((B, TOPK), lambda e, c: (0, 0)),
            pl.BlockSpec((B, H), lambda e, c: (0, 0)),
            pl.BlockSpec((1, CI, H), lambda e, c: (e, 2 * c, 0)),
            pl.BlockSpec((1, CI, H), lambda e, c: (e, 2 * c + 1, 0)),
            pl.BlockSpec((E, I2), lambda e, c: (0, 0)),

            pl.BlockSpec((1, H, CI), lambda e, c: (e, 0, c)),
            pl.BlockSpec((E, H), lambda e, c: (0, 0)),
        ],
        out_specs=pl.BlockSpec((B, TOPK, H), lambda e, c: (0, 0, 0)),
        out_shape=jax.ShapeDtypeStruct((B, TOPK, H), jnp.float32),
        scratch_shapes=[pltpu.VMEM((B, H), jnp.float32)],
        compiler_params=pltpu.CompilerParams(
            dimension_semantics=("arbitrary", "arbitrary")),
    )(idx, t, gate_up_proj, gate_up_proj, gate_up_proj_bias, down_proj,
      down_proj_bias)
    return out
